# baseline (speedup 1.0000x reference)
import functools
import math

import jax
import jax.numpy as jnp
from jax import lax
from jax.experimental import pallas as pl
from jax.experimental.pallas import tpu as pltpu
from jax.experimental.pallas import tpu_sc as plsc

KEEP_RATIO = 0.5
MIN_TOKENS = 1


def _score_mask_body(k, aw_ref, mask_ref, mask16_ref, acc_ref):
    h = pl.program_id(1)
    acc_ref[h, :] = jnp.sum(aw_ref[0, 0], axis=0)

    @pl.when(h == pl.num_programs(1) - 1)
    def _():
        s = jnp.mean(acc_ref[...], axis=0)
        T = s.shape[0]
        s_i = s[:, None]
        s_j = s[None, :]
        i_idx = jax.lax.broadcasted_iota(jnp.int32, (T, T), 0)
        j_idx = jax.lax.broadcasted_iota(jnp.int32, (T, T), 1)
        beats = (s_j > s_i) | ((s_j == s_i) & (j_idx < i_idx))
        rank = jnp.sum(beats.astype(jnp.int32), axis=1)
        pos = jax.lax.broadcasted_iota(jnp.int32, (T,), 0)
        keep = (rank < k) | (pos == 0)
        mask_ref[0, 0, :] = keep.astype(jnp.int32)
        keepf = keep.astype(jnp.float32)
        mask16_ref[0] = jnp.broadcast_to(keepf[:, None], (T, 16))


def _make_sc_prune(B, T, D):
    info = plsc.get_sparse_core_info()
    NC, NS = info.num_cores, info.num_subcores
    NW = NC * NS
    rows_total = B * T
    RPW = rows_total // NW
    WPB = NW // B
    CH = 16
    n_chunks = RPW // CH
    mesh = plsc.VectorSubcoreMesh(core_axis_name="c", subcore_axis_name="s")

    @functools.partial(
        pl.kernel,
        mesh=mesh,
        out_type=jax.ShapeDtypeStruct((B, T, D), jnp.float32),
        scratch_types=[
            pltpu.VMEM((RPW, 16), jnp.float32),
            pltpu.VMEM((CH, D), jnp.float32),
        ],
    )
    def sc_prune(hs_hbm, m16_hbm, out_hbm, mask_v, hbuf):
        wid = lax.axis_index("s") * NC + lax.axis_index("c")
        b = wid // WPB
        t0 = (wid % WPB) * RPW
        pltpu.sync_copy(m16_hbm.at[b, pl.ds(t0, RPW)], mask_v)
        for chunk in range(n_chunks):
            r0 = t0 + chunk * CH
            pltpu.sync_copy(hs_hbm.at[b, pl.ds(r0, CH)], hbuf)

            def row_body(r, carry, _chunk=chunk):
                kv = mask_v[_chunk * CH + r, :]
                for c in range(D // 16):
                    sl = pl.ds(c * 16, 16)
                    hbuf[r, sl] = hbuf[r, sl] * kv
                return carry

            lax.fori_loop(0, CH, row_body, 0)
            pltpu.sync_copy(hbuf, out_hbm.at[b, pl.ds(r0, CH)])

    return sc_prune


@jax.jit
def kernel(hidden_states, attention_weights):
    B, T, D = hidden_states.shape
    _, H, _, _ = attention_weights.shape
    k = min(max(MIN_TOKENS, math.ceil(KEEP_RATIO * T)), T)

    mask_i32, mask16 = pl.pallas_call(
        functools.partial(_score_mask_body, k),
        grid=(B, H),
        in_specs=[pl.BlockSpec((1, 1, T, T), lambda b, h: (b, h, 0, 0))],
        out_specs=[
            pl.BlockSpec((1, 1, T), lambda b, h: (b, 0, 0)),
            pl.BlockSpec((1, T, 16), lambda b, h: (b, 0, 0)),
        ],
        out_shape=[
            jax.ShapeDtypeStruct((B, 1, T), jnp.int32),
            jax.ShapeDtypeStruct((B, T, 16), jnp.float32),
        ],
        scratch_shapes=[pltpu.VMEM((H, T), jnp.float32)],
        compiler_params=pltpu.CompilerParams(
            dimension_semantics=("arbitrary", "arbitrary"),
        ),
    )(attention_weights)

    pruned = _make_sc_prune(B, T, D)(hidden_states, mask16)

    return (pruned, mask_i32.reshape(B, T).astype(bool))


# speedup vs baseline: 1.1941x; 1.1941x over previous
import functools
import math

import jax
import jax.numpy as jnp
from jax.experimental import pallas as pl
from jax.experimental.pallas import tpu as pltpu

KEEP_RATIO = 0.5
MIN_TOKENS = 1


def _score_body(aw_ref, scores_ref, acc_ref, comp_ref):
    h = pl.program_id(1)
    block = aw_ref[0, 0]
    T = block.shape[1]
    G = 8
    rows = block.shape[0] // G

    @pl.when(h == 0)
    def _():
        acc_ref[0, :] = jnp.zeros((T,), jnp.float32)
        comp_ref[0, :] = jnp.zeros((T,), jnp.float32)

    hi = acc_ref[0, :]
    lo = comp_ref[0, :]
    for g in range(G):
        x = jnp.sum(block[g * rows : (g + 1) * rows], axis=0)
        y = x - lo
        t = hi + y
        lo = (t - hi) - y
        hi = t
    acc_ref[0, :] = hi
    comp_ref[0, :] = lo

    @pl.when(h == pl.num_programs(1) - 1)
    def _():
        H = pl.num_programs(1)
        scores_ref[0, 0, :] = hi / jnp.float32(H) - lo / jnp.float32(H)


def _prune_body(k, scores_ref, hs_ref, out_ref, mask_ref):
    s = scores_ref[0, 0, :]
    T = s.shape[0]
    s_i = s[:, None]
    s_j = s[None, :]
    i_idx = jax.lax.broadcasted_iota(jnp.int32, (T, T), 0)
    j_idx = jax.lax.broadcasted_iota(jnp.int32, (T, T), 1)
    beats = (s_j > s_i) | ((s_j == s_i) & (j_idx < i_idx))
    rank = jnp.sum(beats.astype(jnp.int32), axis=1)
    pos = jax.lax.broadcasted_iota(jnp.int32, (T,), 0)
    keep = (rank < k) | (pos == 0)
    mask_ref[0, 0, :] = keep.astype(jnp.int32)
    out_ref[0] = hs_ref[0] * keep.astype(out_ref.dtype)[:, None]


@jax.jit
def kernel(hidden_states, attention_weights):
    B, T, D = hidden_states.shape
    _, H, _, _ = attention_weights.shape
    k = min(max(MIN_TOKENS, math.ceil(KEEP_RATIO * T)), T)

    scores = pl.pallas_call(
        _score_body,
        grid=(B, H),
        in_specs=[pl.BlockSpec((1, 1, T, T), lambda b, h: (b, h, 0, 0))],
        out_specs=pl.BlockSpec((1, 1, T), lambda b, h: (b, 0, 0)),
        out_shape=jax.ShapeDtypeStruct((B, 1, T), jnp.float32),
        scratch_shapes=[
            pltpu.VMEM((1, T), jnp.float32),
            pltpu.VMEM((1, T), jnp.float32),
        ],
        compiler_params=pltpu.CompilerParams(
            dimension_semantics=("arbitrary", "arbitrary"),
        ),
    )(attention_weights)

    pruned, mask_i32 = pl.pallas_call(
        functools.partial(_prune_body, k),
        grid=(B,),
        in_specs=[
            pl.BlockSpec((1, 1, T), lambda b: (b, 0, 0)),
            pl.BlockSpec((1, T, D), lambda b: (b, 0, 0)),
        ],
        out_specs=[
            pl.BlockSpec((1, T, D), lambda b: (b, 0, 0)),
            pl.BlockSpec((1, 1, T), lambda b: (b, 0, 0)),
        ],
        out_shape=[
            jax.ShapeDtypeStruct((B, T, D), hidden_states.dtype),
            jax.ShapeDtypeStruct((B, 1, T), jnp.int32),
        ],
    )(scores, hidden_states)

    return (pruned, mask_i32.reshape(B, T).astype(bool))
